# baseline (speedup 1.0000x reference)
import functools

import jax
import jax.numpy as jnp
from jax import lax
from jax.experimental import pallas as pl
from jax.experimental.pallas import tpu as pltpu
from jax.experimental.pallas import tpu_sc as plsc

D = 4
H = 128
OUT = 16
PACK = 32
LN_EPS = 1e-5

NC = 2
NS = 16
NW = NC * NS
CHUNK = 80


def _pack_body(blk, x_ref, w_ref, ls_ref, o_ref):
    xr = x_ref[...].reshape(blk, D, H)
    xm = (xr[:, 0, :] + xr[:, 1, :] + xr[:, 2, :] + xr[:, 3, :]) * 0.25
    wp = w_ref[...] * ls_ref[...]
    px = jnp.dot(xm, wp, preferred_element_type=jnp.float32)
    sx = jnp.sum(xm, axis=1, keepdims=True)
    qx = jnp.sum(xm * xm, axis=1, keepdims=True)
    pad = jnp.zeros((blk, PACK - OUT - 2), jnp.float32)
    o_ref[...] = jnp.concatenate([px, sx, qx, pad], axis=1)


def _pack_table(x2d, w_half, ls_half, blk):
    n = x2d.shape[0] // D
    return pl.pallas_call(
        functools.partial(_pack_body, blk),
        grid=(n // blk,),
        in_specs=[
            pl.BlockSpec((blk * D, H), lambda i: (i, 0)),
            pl.BlockSpec((H, OUT), lambda i: (0, 0)),
            pl.BlockSpec((H, 1), lambda i: (0, 0)),
        ],
        out_specs=pl.BlockSpec((blk, PACK), lambda i: (i, 0)),
        out_shape=jax.ShapeDtypeStruct((n, PACK), jnp.float32),
    )(x2d, w_half, ls_half)


def _consts_body(w_ref, ls_ref, lb_ref, b_ref, c_ref):
    w = w_ref[...]
    cw = jnp.dot(ls_ref[...], w, preferred_element_type=jnp.float32)
    cb = jnp.dot(lb_ref[...], w, preferred_element_type=jnp.float32) + b_ref[...]
    c_ref[...] = jnp.concatenate([cw, cb], axis=0)


def _make_consts(W, ln_scale, ln_bias, b):
    return pl.pallas_call(
        _consts_body,
        out_shape=jax.ShapeDtypeStruct((2, OUT), jnp.float32),
    )(W, ln_scale.reshape(1, 2 * H), ln_bias.reshape(1, 2 * H), b.reshape(1, OUT))


def _sc_sheaf(tabx, tabe, row, col, consts, n_inc):
    per_w = n_inc // NW
    n_chunk = per_w // CHUNK
    mesh = plsc.VectorSubcoreMesh(core_axis_name="c", subcore_axis_name="s")

    @functools.partial(
        pl.kernel,
        mesh=mesh,
        out_type=jax.ShapeDtypeStruct((n_inc, OUT), jnp.float32),
        compiler_params=pltpu.CompilerParams(
            use_tc_tiling_on_sc=False, needs_layout_passes=False),
        scratch_types=[
            pltpu.VMEM((per_w,), jnp.int32),
            pltpu.VMEM((per_w,), jnp.int32),
            pltpu.VMEM((CHUNK, PACK), jnp.float32),
            pltpu.VMEM((CHUNK, PACK), jnp.float32),
            pltpu.VMEM((CHUNK, PACK), jnp.float32),
            pltpu.VMEM((CHUNK, PACK), jnp.float32),
            pltpu.VMEM((CHUNK, OUT), jnp.float32),
            pltpu.VMEM((CHUNK, OUT), jnp.float32),
            pltpu.VMEM((2, OUT), jnp.float32),
            pltpu.SemaphoreType.DMA,
            pltpu.SemaphoreType.DMA,
            pltpu.SemaphoreType.DMA,
            pltpu.SemaphoreType.DMA,
        ],
    )
    def k(tabx_hbm, tabe_hbm, row_hbm, col_hbm, c_hbm, out_hbm,
          rows_v, cols_v, bxa_v, bea_v, bxb_v, beb_v, oba_v, obb_v,
          cc_v, sema, semb, semoa, semob):
        wid = lax.axis_index("s") * NC + lax.axis_index("c")
        base = wid * per_w
        pltpu.sync_copy(c_hbm, cc_v)
        pltpu.sync_copy(row_hbm.at[pl.ds(base, per_w)], rows_v)
        pltpu.sync_copy(col_hbm.at[pl.ds(base, per_w)], cols_v)
        cw = cc_v[0, :]
        cb = cc_v[1, :]

        def fire(ci, bx_v, be_v, sem):
            off = ci * CHUNK
            cpx = pltpu.async_copy(
                tabx_hbm.at[rows_v.at[pl.ds(off, CHUNK)]], bx_v, sem)
            cpe = pltpu.async_copy(
                tabe_hbm.at[cols_v.at[pl.ds(off, CHUNK)]], be_v, sem)
            return cpx, cpe

        lane0 = jnp.full((16,), 0, jnp.int32)
        lane1 = jnp.full((16,), 1, jnp.int32)

        def drain_out(ob_v, semo):
            pltpu.make_async_copy(
                tabx_hbm.at[pl.ds(0, CHUNK), 0:OUT], ob_v, semo).wait()

        def compute(ci, bx_v, be_v, ob_v, semo):
            @pl.when(ci >= 2)
            def _():
                drain_out(ob_v, semo)

            @plsc.parallel_loop(0, CHUNK, unroll=8)
            def inc_body(j):
                st = bx_v[j, OUT:PACK] + be_v[j, OUT:PACK]
                s = st.at[lane0].get(mode="promise_in_bounds")
                q = st.at[lane1].get(mode="promise_in_bounds")
                mu = s * (1.0 / (2 * H))
                v = q * (1.0 / (2 * H)) - mu * mu + LN_EPS
                iv = plsc.bitcast(v, jnp.int32)
                iv = 0x5F3759DF - lax.shift_right_arithmetic(iv, 1)
                y = plsc.bitcast(iv, jnp.float32)
                hv = 0.5 * v
                y = y * (1.5 - hv * y * y)
                y = y * (1.5 - hv * y * y)
                dv = bx_v[j, 0:OUT] + be_v[j, 0:OUT]
                t = dv * y - (mu * y) * cw + cb
                ob_v[j, :] = 1.0 / (1.0 + jnp.exp(-t))

            pltpu.async_copy(
                ob_v, out_hbm.at[pl.ds(base + ci * CHUNK, CHUNK)], semo)

        def drain(bx_v, be_v, sem):
            pltpu.make_async_copy(tabx_hbm.at[pl.ds(0, CHUNK)], bx_v, sem).wait()
            pltpu.make_async_copy(tabe_hbm.at[pl.ds(0, CHUNK)], be_v, sem).wait()

        fire(0, bxa_v, bea_v, sema)

        def pair_body(p, _):
            fire(2 * p + 1, bxb_v, beb_v, semb)
            drain(bxa_v, bea_v, sema)
            compute(2 * p, bxa_v, bea_v, oba_v, semoa)
            fire(2 * p + 2, bxa_v, bea_v, sema)
            drain(bxb_v, beb_v, semb)
            compute(2 * p + 1, bxb_v, beb_v, obb_v, semob)
            return 0

        lax.fori_loop(0, (n_chunk - 1) // 2, pair_body, 0)
        drain(bxa_v, bea_v, sema)
        compute(n_chunk - 1, bxa_v, bea_v, oba_v, semoa)
        drain_out(oba_v, semoa)
        drain_out(obb_v, semob)

    return k(tabx, tabe, row, col, consts)


def kernel(x, e, hyperedge_index, node_types, hyperedge_types,
           ln_scale, ln_bias, W, b):
    n_nodes = x.shape[0] // D
    n_edges = e.shape[0] // D
    n_inc = hyperedge_index.shape[1]

    tabx = _pack_table(x, W[:H], ln_scale[:H, None], 1000)
    tabe = _pack_table(e, W[H:], ln_scale[H:, None], 1000)
    consts = _make_consts(W, ln_scale, ln_bias, b)

    row = hyperedge_index[0].astype(jnp.int32)
    col = hyperedge_index[1].astype(jnp.int32)

    return _sc_sheaf(tabx, tabe, row, col, consts, n_inc)


# speedup vs baseline: 1.6105x; 1.6105x over previous
import functools

import jax
import jax.numpy as jnp
from jax import lax
from jax.experimental import pallas as pl
from jax.experimental.pallas import tpu as pltpu
from jax.experimental.pallas import tpu_sc as plsc

D = 4
H = 128
OUT = 16
PACK = 32
LN_EPS = 1e-5

NC = 2
NS = 16
NW = NC * NS
CHUNK = 80


def _pack_body(blk, x_ref, w_ref, ls_ref, o_ref):
    xr = x_ref[...].reshape(blk, D, H)
    xm = (xr[:, 0, :] + xr[:, 1, :] + xr[:, 2, :] + xr[:, 3, :]) * 0.25
    wp = w_ref[...] * ls_ref[...]
    px = jnp.dot(xm, wp, preferred_element_type=jnp.float32)
    sx = jnp.sum(xm, axis=1, keepdims=True)
    qx = jnp.sum(xm * xm, axis=1, keepdims=True)
    pad = jnp.zeros((blk, PACK - OUT - 2), jnp.float32)
    o_ref[...] = jnp.concatenate([px, sx, qx, pad], axis=1)


def _pack_table(x2d, w_half, ls_half, blk):
    n = x2d.shape[0] // D
    return pl.pallas_call(
        functools.partial(_pack_body, blk),
        grid=(n // blk,),
        in_specs=[
            pl.BlockSpec((blk * D, H), lambda i: (i, 0)),
            pl.BlockSpec((H, OUT), lambda i: (0, 0)),
            pl.BlockSpec((H, 1), lambda i: (0, 0)),
        ],
        out_specs=pl.BlockSpec((blk, PACK), lambda i: (i, 0)),
        out_shape=jax.ShapeDtypeStruct((n, PACK), jnp.float32),
    )(x2d, w_half, ls_half)


def _consts_body(w_ref, ls_ref, lb_ref, b_ref, c_ref):
    w = w_ref[...]
    cw = jnp.dot(ls_ref[...], w, preferred_element_type=jnp.float32)
    cb = jnp.dot(lb_ref[...], w, preferred_element_type=jnp.float32) + b_ref[...]
    c_ref[...] = jnp.concatenate([cw, cb], axis=0)


def _make_consts(W, ln_scale, ln_bias, b):
    return pl.pallas_call(
        _consts_body,
        out_shape=jax.ShapeDtypeStruct((2, OUT), jnp.float32),
    )(W, ln_scale.reshape(1, 2 * H), ln_bias.reshape(1, 2 * H), b.reshape(1, OUT))


def _sc_sheaf(tabx, tabe, row, col, consts, n_inc):
    per_w = n_inc // NW
    n_chunk = per_w // CHUNK
    mesh = plsc.VectorSubcoreMesh(core_axis_name="c", subcore_axis_name="s")

    @functools.partial(
        pl.kernel,
        mesh=mesh,
        out_type=jax.ShapeDtypeStruct((OUT, n_inc), jnp.float32),
        compiler_params=pltpu.CompilerParams(
            use_tc_tiling_on_sc=False, needs_layout_passes=False),
        scratch_types=[
            pltpu.VMEM((per_w,), jnp.int32),
            pltpu.VMEM((per_w,), jnp.int32),
            pltpu.VMEM((CHUNK, PACK), jnp.float32),
            pltpu.VMEM((CHUNK, PACK), jnp.float32),
            pltpu.VMEM((CHUNK, PACK), jnp.float32),
            pltpu.VMEM((CHUNK, PACK), jnp.float32),
            pltpu.VMEM((OUT, CHUNK), jnp.float32),
            pltpu.VMEM((OUT, CHUNK), jnp.float32),
            pltpu.VMEM((2, OUT), jnp.float32),
            pltpu.SemaphoreType.DMA,
            pltpu.SemaphoreType.DMA,
            pltpu.SemaphoreType.DMA,
            pltpu.SemaphoreType.DMA,
        ],
    )
    def k(tabx_hbm, tabe_hbm, row_hbm, col_hbm, c_hbm, out_hbm,
          rows_v, cols_v, bxa_v, bea_v, bxb_v, beb_v, oba_v, obb_v,
          cc_v, sema, semb, semoa, semob):
        wid = lax.axis_index("s") * NC + lax.axis_index("c")
        base = wid * per_w
        pltpu.sync_copy(c_hbm, cc_v)
        pltpu.sync_copy(row_hbm.at[pl.ds(base, per_w)], rows_v)
        pltpu.sync_copy(col_hbm.at[pl.ds(base, per_w)], cols_v)
        cw = cc_v[0, :]
        cb = cc_v[1, :]

        def fire(ci, bx_v, be_v, sem):
            off = ci * CHUNK
            cpx = pltpu.async_copy(
                tabx_hbm.at[rows_v.at[pl.ds(off, CHUNK)]], bx_v, sem)
            cpe = pltpu.async_copy(
                tabe_hbm.at[cols_v.at[pl.ds(off, CHUNK)]], be_v, sem)
            return cpx, cpe

        lane0 = jnp.full((16,), 0, jnp.int32)
        lane1 = jnp.full((16,), 1, jnp.int32)
        iota16 = lax.iota(jnp.int32, 16)

        def drain_out(ob_v, semo):
            pltpu.make_async_copy(
                out_hbm.at[:, pl.ds(0, CHUNK)], ob_v, semo).wait()

        def compute(ci, bx_v, be_v, ob_v, semo):
            @pl.when(ci >= 2)
            def _():
                drain_out(ob_v, semo)

            @plsc.parallel_loop(0, CHUNK, unroll=8)
            def inc_body(j):
                st = bx_v[j, OUT:PACK] + be_v[j, OUT:PACK]
                s = st.at[lane0].get(mode="promise_in_bounds")
                q = st.at[lane1].get(mode="promise_in_bounds")
                mu = s * (1.0 / (2 * H))
                v = q * (1.0 / (2 * H)) - mu * mu + LN_EPS
                iv = plsc.bitcast(v, jnp.int32)
                iv = 0x5F3759DF - lax.shift_right_arithmetic(iv, 1)
                y = plsc.bitcast(iv, jnp.float32)
                hv = 0.5 * v
                y = y * (1.5 - hv * y * y)
                y = y * (1.5 - hv * y * y)
                dv = bx_v[j, 0:OUT] + be_v[j, 0:OUT]
                t = dv * y - (mu * y) * cw + cb
                r = 1.0 / (1.0 + jnp.exp(-t))
                plsc.store_scatter(ob_v, [iota16, jnp.full((16,), j)], r)

            pltpu.async_copy(
                ob_v, out_hbm.at[:, pl.ds(base + ci * CHUNK, CHUNK)], semo)

        def drain(bx_v, be_v, sem):
            pltpu.make_async_copy(tabx_hbm.at[pl.ds(0, CHUNK)], bx_v, sem).wait()
            pltpu.make_async_copy(tabe_hbm.at[pl.ds(0, CHUNK)], be_v, sem).wait()

        fire(0, bxa_v, bea_v, sema)

        def pair_body(p, _):
            fire(2 * p + 1, bxb_v, beb_v, semb)
            drain(bxa_v, bea_v, sema)
            compute(2 * p, bxa_v, bea_v, oba_v, semoa)
            fire(2 * p + 2, bxa_v, bea_v, sema)
            drain(bxb_v, beb_v, semb)
            compute(2 * p + 1, bxb_v, beb_v, obb_v, semob)
            return 0

        lax.fori_loop(0, (n_chunk - 1) // 2, pair_body, 0)
        drain(bxa_v, bea_v, sema)
        compute(n_chunk - 1, bxa_v, bea_v, oba_v, semoa)
        drain_out(oba_v, semoa)
        drain_out(obb_v, semob)

    return k(tabx, tabe, row, col, consts)


def kernel(x, e, hyperedge_index, node_types, hyperedge_types,
           ln_scale, ln_bias, W, b):
    n_nodes = x.shape[0] // D
    n_edges = e.shape[0] // D
    n_inc = hyperedge_index.shape[1]

    tabx = _pack_table(x, W[:H], ln_scale[:H, None], 1000)
    tabe = _pack_table(e, W[H:], ln_scale[H:, None], 1000)
    consts = _make_consts(W, ln_scale, ln_bias, b)

    row = hyperedge_index[0].astype(jnp.int32)
    col = hyperedge_index[1].astype(jnp.int32)

    out_t = _sc_sheaf(tabx, tabe, row, col, consts, n_inc)
    return jnp.swapaxes(out_t, 0, 1)


# speedup vs baseline: 1.6910x; 1.0500x over previous
import functools

import jax
import jax.numpy as jnp
from jax import lax
from jax.experimental import pallas as pl
from jax.experimental.pallas import tpu as pltpu
from jax.experimental.pallas import tpu_sc as plsc

D = 4
H = 128
OUT = 16
PACK = 32
LN_EPS = 1e-5

NC = 2
NS = 16
NW = NC * NS
CHUNK = 80


def _pack_body(blk, x_ref, w_ref, ls_ref, o_ref):
    xr = x_ref[...].reshape(blk, D, H)
    xm = (xr[:, 0, :] + xr[:, 1, :] + xr[:, 2, :] + xr[:, 3, :]) * 0.25
    wp = w_ref[...] * ls_ref[...]
    px = jnp.dot(xm, wp, preferred_element_type=jnp.float32)
    sx = jnp.sum(xm, axis=1, keepdims=True)
    qx = jnp.sum(xm * xm, axis=1, keepdims=True)
    pad = jnp.zeros((blk, PACK - OUT - 2), jnp.float32)
    o_ref[...] = jnp.concatenate([px, sx, qx, pad], axis=1)


def _pack_table(x2d, w_half, ls_half, blk):
    n = x2d.shape[0] // D
    return pl.pallas_call(
        functools.partial(_pack_body, blk),
        grid=(n // blk,),
        in_specs=[
            pl.BlockSpec((blk * D, H), lambda i: (i, 0)),
            pl.BlockSpec((H, OUT), lambda i: (0, 0)),
            pl.BlockSpec((H, 1), lambda i: (0, 0)),
        ],
        out_specs=pl.BlockSpec((blk, PACK), lambda i: (i, 0)),
        out_shape=jax.ShapeDtypeStruct((n, PACK), jnp.float32),
    )(x2d, w_half, ls_half)


def _consts_body(w_ref, ls_ref, lb_ref, b_ref, c_ref):
    w = w_ref[...]
    cw = jnp.dot(ls_ref[...], w, preferred_element_type=jnp.float32)
    cb = jnp.dot(lb_ref[...], w, preferred_element_type=jnp.float32) + b_ref[...]
    c_ref[...] = jnp.concatenate([cw, cb], axis=0)


def _make_consts(W, ln_scale, ln_bias, b):
    return pl.pallas_call(
        _consts_body,
        out_shape=jax.ShapeDtypeStruct((2, OUT), jnp.float32),
    )(W, ln_scale.reshape(1, 2 * H), ln_bias.reshape(1, 2 * H), b.reshape(1, OUT))


def _sc_sheaf(tabx, tabe, hi, consts, n_inc):
    per_w = n_inc // NW
    n_chunk = per_w // CHUNK
    mesh = plsc.VectorSubcoreMesh(core_axis_name="c", subcore_axis_name="s")

    @functools.partial(
        pl.kernel,
        mesh=mesh,
        out_type=jax.ShapeDtypeStruct((OUT, n_inc), jnp.float32),
        compiler_params=pltpu.CompilerParams(
            use_tc_tiling_on_sc=False, needs_layout_passes=False),
        scratch_types=[
            pltpu.VMEM((per_w,), jnp.int32),
            pltpu.VMEM((per_w,), jnp.int32),
            pltpu.VMEM((CHUNK, PACK), jnp.float32),
            pltpu.VMEM((CHUNK, PACK), jnp.float32),
            pltpu.VMEM((CHUNK, PACK), jnp.float32),
            pltpu.VMEM((CHUNK, PACK), jnp.float32),
            pltpu.VMEM((OUT, CHUNK), jnp.float32),
            pltpu.VMEM((OUT, CHUNK), jnp.float32),
            pltpu.VMEM((2, OUT), jnp.float32),
            pltpu.SemaphoreType.DMA,
            pltpu.SemaphoreType.DMA,
            pltpu.SemaphoreType.DMA,
            pltpu.SemaphoreType.DMA,
        ],
    )
    def k(tabx_hbm, tabe_hbm, hi_hbm, c_hbm, out_hbm,
          rows_v, cols_v, bxa_v, bea_v, bxb_v, beb_v, oba_v, obb_v,
          cc_v, sema, semb, semoa, semob):
        wid = lax.axis_index("s") * NC + lax.axis_index("c")
        base = wid * per_w
        pltpu.sync_copy(c_hbm, cc_v)
        pltpu.sync_copy(hi_hbm.at[0, pl.ds(base, per_w)], rows_v)
        pltpu.sync_copy(hi_hbm.at[1, pl.ds(base, per_w)], cols_v)
        cw = cc_v[0, :]
        cb = cc_v[1, :]

        def fire(ci, bx_v, be_v, sem):
            off = ci * CHUNK
            cpx = pltpu.async_copy(
                tabx_hbm.at[rows_v.at[pl.ds(off, CHUNK)]], bx_v, sem)
            cpe = pltpu.async_copy(
                tabe_hbm.at[cols_v.at[pl.ds(off, CHUNK)]], be_v, sem)
            return cpx, cpe

        lane0 = jnp.full((16,), 0, jnp.int32)
        lane1 = jnp.full((16,), 1, jnp.int32)
        iota16 = lax.iota(jnp.int32, 16)

        def drain_out(ob_v, semo):
            pltpu.make_async_copy(
                out_hbm.at[:, pl.ds(0, CHUNK)], ob_v, semo).wait()

        def compute(ci, bx_v, be_v, ob_v, semo):
            @pl.when(ci >= 2)
            def _():
                drain_out(ob_v, semo)

            @plsc.parallel_loop(0, CHUNK, unroll=8)
            def inc_body(j):
                st = bx_v[j, OUT:PACK] + be_v[j, OUT:PACK]
                s = st.at[lane0].get(mode="promise_in_bounds")
                q = st.at[lane1].get(mode="promise_in_bounds")
                mu = s * (1.0 / (2 * H))
                v = q * (1.0 / (2 * H)) - mu * mu + LN_EPS
                iv = plsc.bitcast(v, jnp.int32)
                iv = 0x5F3759DF - lax.shift_right_arithmetic(iv, 1)
                y = plsc.bitcast(iv, jnp.float32)
                hv = 0.5 * v
                y = y * (1.5 - hv * y * y)
                y = y * (1.5 - hv * y * y)
                dv = bx_v[j, 0:OUT] + be_v[j, 0:OUT]
                t = dv * y - (mu * y) * cw + cb
                r = 1.0 / (1.0 + jnp.exp(-t))
                plsc.store_scatter(ob_v, [iota16, jnp.full((16,), j)], r)

            pltpu.async_copy(
                ob_v, out_hbm.at[:, pl.ds(base + ci * CHUNK, CHUNK)], semo)

        def drain(bx_v, be_v, sem):
            pltpu.make_async_copy(tabx_hbm.at[pl.ds(0, CHUNK)], bx_v, sem).wait()
            pltpu.make_async_copy(tabe_hbm.at[pl.ds(0, CHUNK)], be_v, sem).wait()

        fire(0, bxa_v, bea_v, sema)

        def pair_body(p, _):
            fire(2 * p + 1, bxb_v, beb_v, semb)
            drain(bxa_v, bea_v, sema)
            compute(2 * p, bxa_v, bea_v, oba_v, semoa)
            fire(2 * p + 2, bxa_v, bea_v, sema)
            drain(bxb_v, beb_v, semb)
            compute(2 * p + 1, bxb_v, beb_v, obb_v, semob)
            return 0

        lax.fori_loop(0, (n_chunk - 1) // 2, pair_body, 0)
        drain(bxa_v, bea_v, sema)
        compute(n_chunk - 1, bxa_v, bea_v, oba_v, semoa)
        drain_out(oba_v, semoa)
        drain_out(obb_v, semob)

    return k(tabx, tabe, hi, consts)


def kernel(x, e, hyperedge_index, node_types, hyperedge_types,
           ln_scale, ln_bias, W, b):
    n_nodes = x.shape[0] // D
    n_edges = e.shape[0] // D
    n_inc = hyperedge_index.shape[1]

    tabx = _pack_table(x, W[:H], ln_scale[:H, None], 2000)
    tabe = _pack_table(e, W[H:], ln_scale[H:, None], 1000)
    consts = _make_consts(W, ln_scale, ln_bias, b)

    hi = hyperedge_index.astype(jnp.int32)

    out_t = _sc_sheaf(tabx, tabe, hi, consts, n_inc)
    return jnp.swapaxes(out_t, 0, 1)


# speedup vs baseline: 1.7303x; 1.0232x over previous
import functools

import jax
import jax.numpy as jnp
from jax import lax
from jax.experimental import pallas as pl
from jax.experimental.pallas import tpu as pltpu
from jax.experimental.pallas import tpu_sc as plsc

D = 4
H = 128
OUT = 16
PACK = 32
LN_EPS = 1e-5

NC = 2
NS = 16
NW = NC * NS
CHUNK = 80


def _pack_body(blk, x_ref, w_ref, ls_ref, o_ref):
    xr = x_ref[...].reshape(blk, D, H)
    xm = (xr[:, 0, :] + xr[:, 1, :] + xr[:, 2, :] + xr[:, 3, :]) * 0.25
    wp = w_ref[...] * ls_ref[...]
    px = jnp.dot(xm, wp, preferred_element_type=jnp.float32)
    sx = jnp.sum(xm, axis=1, keepdims=True)
    qx = jnp.sum(xm * xm, axis=1, keepdims=True)
    pad = jnp.zeros((blk, PACK - OUT - 2), jnp.float32)
    o_ref[...] = jnp.concatenate([px, sx, qx, pad], axis=1)


def _pack_table(x2d, w_half, ls_half, blk):
    n = x2d.shape[0] // D
    return pl.pallas_call(
        functools.partial(_pack_body, blk),
        grid=(n // blk,),
        in_specs=[
            pl.BlockSpec((blk * D, H), lambda i: (i, 0)),
            pl.BlockSpec((H, OUT), lambda i: (0, 0)),
            pl.BlockSpec((H, 1), lambda i: (0, 0)),
        ],
        out_specs=pl.BlockSpec((blk, PACK), lambda i: (i, 0)),
        out_shape=jax.ShapeDtypeStruct((n, PACK), jnp.float32),
    )(x2d, w_half, ls_half)


def _consts_body(w_ref, ls_ref, lb_ref, b_ref, c_ref):
    w = w_ref[...]
    cw = jnp.dot(ls_ref[...], w, preferred_element_type=jnp.float32)
    cb = jnp.dot(lb_ref[...], w, preferred_element_type=jnp.float32) + b_ref[...]
    c_ref[...] = jnp.concatenate([cw, cb], axis=0)


def _make_consts(W, ln_scale, ln_bias, b):
    return pl.pallas_call(
        _consts_body,
        out_shape=jax.ShapeDtypeStruct((2, OUT), jnp.float32),
    )(W, ln_scale.reshape(1, 2 * H), ln_bias.reshape(1, 2 * H), b.reshape(1, OUT))


def _sc_sheaf(tabx, tabe, hi, consts, n_inc):
    per_w = n_inc // NW
    n_chunk = per_w // CHUNK
    mesh = plsc.VectorSubcoreMesh(core_axis_name="c", subcore_axis_name="s")

    @functools.partial(
        pl.kernel,
        mesh=mesh,
        out_type=jax.ShapeDtypeStruct((OUT, n_inc), jnp.float32),
        compiler_params=pltpu.CompilerParams(
            use_tc_tiling_on_sc=False, needs_layout_passes=False),
        scratch_types=[
            pltpu.VMEM((per_w,), jnp.int32),
            pltpu.VMEM((per_w,), jnp.int32),
            pltpu.VMEM((CHUNK, PACK), jnp.float32),
            pltpu.VMEM((CHUNK, PACK), jnp.float32),
            pltpu.VMEM((CHUNK, PACK), jnp.float32),
            pltpu.VMEM((CHUNK, PACK), jnp.float32),
            pltpu.VMEM((OUT, CHUNK), jnp.float32),
            pltpu.VMEM((OUT, CHUNK), jnp.float32),
            pltpu.VMEM((2, OUT), jnp.float32),
            pltpu.SemaphoreType.DMA,
            pltpu.SemaphoreType.DMA,
            pltpu.SemaphoreType.DMA,
            pltpu.SemaphoreType.DMA,
        ],
    )
    def k(tabx_hbm, tabe_hbm, hi_hbm, c_hbm, out_hbm,
          rows_v, cols_v, bxa_v, bea_v, bxb_v, beb_v, oba_v, obb_v,
          cc_v, sema, semb, semoa, semob):
        wid = lax.axis_index("s") * NC + lax.axis_index("c")
        base = wid * per_w
        pltpu.sync_copy(c_hbm, cc_v)
        pltpu.sync_copy(hi_hbm.at[0, pl.ds(base, per_w)], rows_v)
        pltpu.sync_copy(hi_hbm.at[1, pl.ds(base, per_w)], cols_v)
        cw = cc_v[0, :]
        cb = cc_v[1, :]

        def fire(ci, bx_v, be_v, sem):
            off = ci * CHUNK
            cpx = pltpu.async_copy(
                tabx_hbm.at[rows_v.at[pl.ds(off, CHUNK)]], bx_v, sem)
            cpe = pltpu.async_copy(
                tabe_hbm.at[cols_v.at[pl.ds(off, CHUNK)]], be_v, sem)
            return cpx, cpe

        lane0 = jnp.full((16,), 0, jnp.int32)
        lane1 = jnp.full((16,), 1, jnp.int32)
        iota16 = lax.iota(jnp.int32, 16)

        def drain_out(ob_v, semo):
            pltpu.make_async_copy(
                out_hbm.at[:, pl.ds(0, CHUNK)], ob_v, semo).wait()

        def compute(ci, bx_v, be_v, ob_v, semo):
            @pl.when(ci >= 2)
            def _():
                drain_out(ob_v, semo)

            @plsc.parallel_loop(0, CHUNK, unroll=8)
            def inc_body(j):
                st = bx_v[j, OUT:PACK] + be_v[j, OUT:PACK]
                s = st.at[lane0].get(mode="promise_in_bounds")
                q = st.at[lane1].get(mode="promise_in_bounds")
                mu = s * (1.0 / (2 * H))
                v = q * (1.0 / (2 * H)) - mu * mu + LN_EPS
                iv = plsc.bitcast(v, jnp.int32)
                iv = 0x5F3759DF - lax.shift_right_arithmetic(iv, 1)
                y = plsc.bitcast(iv, jnp.float32)
                hv = 0.5 * v
                y = y * (1.5 - hv * y * y)
                dv = bx_v[j, 0:OUT] + be_v[j, 0:OUT]
                t = (dv - mu * cw) * y + cb
                r = 1.0 / (1.0 + jnp.exp(-t))
                plsc.store_scatter(ob_v, [iota16, jnp.full((16,), j)], r)

            pltpu.async_copy(
                ob_v, out_hbm.at[:, pl.ds(base + ci * CHUNK, CHUNK)], semo)

        def drain(bx_v, be_v, sem):
            pltpu.make_async_copy(tabx_hbm.at[pl.ds(0, CHUNK)], bx_v, sem).wait()
            pltpu.make_async_copy(tabe_hbm.at[pl.ds(0, CHUNK)], be_v, sem).wait()

        fire(0, bxa_v, bea_v, sema)

        def pair_body(p, _):
            fire(2 * p + 1, bxb_v, beb_v, semb)
            drain(bxa_v, bea_v, sema)
            compute(2 * p, bxa_v, bea_v, oba_v, semoa)
            fire(2 * p + 2, bxa_v, bea_v, sema)
            drain(bxb_v, beb_v, semb)
            compute(2 * p + 1, bxb_v, beb_v, obb_v, semob)
            return 0

        lax.fori_loop(0, (n_chunk - 1) // 2, pair_body, 0)
        drain(bxa_v, bea_v, sema)
        compute(n_chunk - 1, bxa_v, bea_v, oba_v, semoa)
        drain_out(oba_v, semoa)
        drain_out(obb_v, semob)

    return k(tabx, tabe, hi, consts)


def kernel(x, e, hyperedge_index, node_types, hyperedge_types,
           ln_scale, ln_bias, W, b):
    n_nodes = x.shape[0] // D
    n_edges = e.shape[0] // D
    n_inc = hyperedge_index.shape[1]

    tabx = _pack_table(x, W[:H], ln_scale[:H, None], 2000)
    tabe = _pack_table(e, W[H:], ln_scale[H:, None], 1000)
    consts = _make_consts(W, ln_scale, ln_bias, b)

    hi = hyperedge_index.astype(jnp.int32)

    out_t = _sc_sheaf(tabx, tabe, hi, consts, n_inc)
    return jnp.swapaxes(out_t, 0, 1)
